# baseline (speedup 1.0000x reference)
import functools

import jax
import jax.numpy as jnp
from jax import lax
from jax.experimental import pallas as pl
from jax.experimental.pallas import tpu as pltpu
from jax.experimental.pallas import tpu_sc as plsc

N = 10000
E = 5000
D = 256
HF = 128
NP = 10240
EP = 5120
NNZ = 160000
NNZ_P = 163840
PAD = NNZ_P - NNZ
CH = 128
NC, NS = 2, 16
EPT = NNZ_P // NS
F32 = jnp.float32


def _mm_body(x_ref, w_ref, o_ref):
    o_ref[...] = jnp.dot(x_ref[...], w_ref[...], preferred_element_type=F32)


def _matmul(x, w, bm=1024):
    m = x.shape[0]
    return pl.pallas_call(
        _mm_body,
        grid=(m // bm,),
        in_specs=[pl.BlockSpec((bm, D), lambda i: (i, 0)),
                  pl.BlockSpec((D, D), lambda i: (0, 0))],
        out_specs=pl.BlockSpec((bm, D), lambda i: (i, 0)),
        out_shape=jax.ShapeDtypeStruct((m, D), F32),
    )(x, w)


def _edge_stage(et0, et1, deg_e, w, a, want_e, bm=1024):
    m = et0.shape[0]

    def body(et0_ref, et1_ref, deg_ref, w_ref, a_ref, *out_refs):
        cnt = deg_ref[:, 0:1]
        s = jnp.where(cnt > 0.0, 1.0 / cnt, 0.0)
        al = a_ref[0, 0]
        e0 = et0_ref[...] * s
        e0 = jnp.where(e0 >= 0.0, e0, al * e0)
        e1 = et1_ref[...] * s
        e1 = jnp.where(e1 >= 0.0, e1, al * e1)
        xn = (jnp.dot(e0, w_ref[:HF, :], preferred_element_type=F32)
              + jnp.dot(e1, w_ref[HF:, :], preferred_element_type=F32))
        out_refs[0][...] = xn
        if want_e:
            out_refs[1][:, :HF] = e0
            out_refs[1][:, HF:] = e1

    n_out = 2 if want_e else 1
    return pl.pallas_call(
        body,
        grid=(m // bm,),
        in_specs=[pl.BlockSpec((bm, HF), lambda i: (i, 0)),
                  pl.BlockSpec((bm, HF), lambda i: (i, 0)),
                  pl.BlockSpec((bm, HF), lambda i: (i, 0)),
                  pl.BlockSpec((D, D), lambda i: (0, 0)),
                  pl.BlockSpec(memory_space=pltpu.SMEM)],
        out_specs=[pl.BlockSpec((bm, D), lambda i: (i, 0))] * n_out,
        out_shape=[jax.ShapeDtypeStruct((m, D), F32)] * n_out,
    )(et0, et1, deg_e, w, a)


def _front(x, w_n2e, w_e2n, a, bm=1024):
    m = x.shape[0]

    def body(x_ref, w1_ref, w2_ref, a_ref, xe_ref, xs_ref):
        al = a_ref[0, 0]
        xe = jnp.dot(x_ref[...], w1_ref[...], preferred_element_type=F32)
        xe_ref[...] = xe
        p = jnp.where(xe >= 0.0, xe, al * xe)
        xs_ref[...] = jnp.dot(p, w2_ref[...], preferred_element_type=F32)

    return pl.pallas_call(
        body,
        grid=(m // bm,),
        in_specs=[pl.BlockSpec((bm, D), lambda i: (i, 0)),
                  pl.BlockSpec((D, D), lambda i: (0, 0)),
                  pl.BlockSpec((D, D), lambda i: (0, 0)),
                  pl.BlockSpec(memory_space=pltpu.SMEM)],
        out_specs=[pl.BlockSpec((bm, D), lambda i: (i, 0))] * 2,
        out_shape=[jax.ShapeDtypeStruct((m, D), F32)] * 2,
    )(x, w_n2e, w_e2n, a)


def _mid(n0, n1, xn_self, deg_n, a_h, w_n2e, w_e2n, a, bm=1024):
    m = n0.shape[0]

    def body(n0_ref, n1_ref, xs_ref, deg_ref, ah_ref, w1_ref, w2_ref, a_ref,
             xe_ref, xs2_ref):
        sc = 1.0 / (deg_ref[:, 0:1] + 1.0)
        ah = ah_ref[0, 0]
        al = a_ref[0, 0]
        t0 = (n0_ref[...] + xs_ref[:, :HF]) * sc
        t0 = jnp.where(t0 >= 0.0, t0, ah * t0)
        t1 = (n1_ref[...] + xs_ref[:, HF:]) * sc
        t1 = jnp.where(t1 >= 0.0, t1, ah * t1)
        xe = (jnp.dot(t0, w1_ref[:HF, :], preferred_element_type=F32)
              + jnp.dot(t1, w1_ref[HF:, :], preferred_element_type=F32))
        xe_ref[...] = xe
        p = jnp.where(xe >= 0.0, xe, al * xe)
        xs2_ref[...] = jnp.dot(p, w2_ref[...], preferred_element_type=F32)

    return pl.pallas_call(
        body,
        grid=(m // bm,),
        in_specs=[pl.BlockSpec((bm, HF), lambda i: (i, 0)),
                  pl.BlockSpec((bm, HF), lambda i: (i, 0)),
                  pl.BlockSpec((bm, D), lambda i: (i, 0)),
                  pl.BlockSpec((bm, HF), lambda i: (i, 0)),
                  pl.BlockSpec(memory_space=pltpu.SMEM),
                  pl.BlockSpec((D, D), lambda i: (0, 0)),
                  pl.BlockSpec((D, D), lambda i: (0, 0)),
                  pl.BlockSpec(memory_space=pltpu.SMEM)],
        out_specs=[pl.BlockSpec((bm, D), lambda i: (i, 0))] * 2,
        out_shape=[jax.ShapeDtypeStruct((m, D), F32)] * 2,
    )(n0, n1, xn_self, deg_n, a_h, w_n2e, w_e2n, a)


def _node_stage(n0, n1, xn_self, deg_n, a_h, bm=1024):
    m = n0.shape[0]

    def body(n0_ref, n1_ref, xs_ref, deg_ref, a_ref, h_ref):
        s = 1.0 / (deg_ref[:, 0:1] + 1.0)
        ah = a_ref[0, 0]
        t0 = (n0_ref[...] + xs_ref[:, :HF]) * s
        t0 = jnp.where(t0 >= 0.0, t0, ah * t0)
        t1 = (n1_ref[...] + xs_ref[:, HF:]) * s
        t1 = jnp.where(t1 >= 0.0, t1, ah * t1)
        h_ref[:, :HF] = t0
        h_ref[:, HF:] = t1

    return pl.pallas_call(
        body,
        grid=(m // bm,),
        in_specs=[pl.BlockSpec((bm, HF), lambda i: (i, 0)),
                  pl.BlockSpec((bm, HF), lambda i: (i, 0)),
                  pl.BlockSpec((bm, D), lambda i: (i, 0)),
                  pl.BlockSpec((bm, HF), lambda i: (i, 0)),
                  pl.BlockSpec(memory_space=pltpu.SMEM)],
        out_specs=pl.BlockSpec((bm, D), lambda i: (i, 0)),
        out_shape=jax.ShapeDtypeStruct((m, D), F32),
    )(n0, n1, xn_self, deg_n, a_h)


@functools.cache
def _mesh():
    return plsc.VectorSubcoreMesh(core_axis_name="c", subcore_axis_name="s",
                                  num_cores=NC, num_subcores=NS)


NCHT = EPT // CH


def _sc_aggregate(table2, src2, dst, rows_out):
    rpt = rows_out // NS
    NH = NCHT // 2

    nbuf = 2

    def body(table2_r, src2_r, dst_r, zeros_r, out, src_b, dst_b, *rest):
        bufs, sems = rest[:nbuf], rest[nbuf + 1:]
        acc = rest[nbuf]
        c = lax.axis_index("c")
        s = lax.axis_index("s")
        r0 = s * rpt
        pltpu.sync_copy(zeros_r.at[pl.ds(r0, rpt)], acc.at[pl.ds(r0, rpt)])
        plsc.subcore_barrier()

        def gather(k, buf, sem):
            pltpu.async_copy(table2_r.at[src_b.at[k]], buf, sem)

        def gwait(buf, sem):
            pltpu.make_async_copy(table2_r.at[src_b.at[0]], buf, sem).wait()

        for h in range(2):
            srow = (c * NNZ_P + s * EPT) // CH + h * NH
            pltpu.sync_copy(
                src2_r.at[pl.ds(pl.multiple_of(srow, 8), NH)], src_b)
            pltpu.sync_copy(
                dst_r.at[pl.ds(pl.multiple_of(s * NCHT + h * NH, 8), NH)],
                dst_b)
            for j in range(nbuf - 1):
                gather(j, bufs[j], sems[j])

            def step(i, carry):
                k0 = i * nbuf
                for j in range(nbuf):
                    gwait(bufs[j], sems[j])
                    pltpu.sync_copy(bufs[j], acc.at[dst_b.at[k0 + j]],
                                    add=True)
                    gather(k0 + j + nbuf - 1, bufs[(j + nbuf - 1) % nbuf],
                           sems[(j + nbuf - 1) % nbuf])
                return carry

            lax.fori_loop(0, NH // nbuf - 1, step, 0)
            k0 = NH - nbuf
            gather(NH - 1, bufs[(nbuf - 1) % nbuf], sems[(nbuf - 1) % nbuf])
            for j in range(nbuf):
                gwait(bufs[j], sems[j])
                pltpu.sync_copy(bufs[j], acc.at[dst_b.at[k0 + j]], add=True)
        plsc.subcore_barrier()
        pltpu.sync_copy(acc.at[pl.ds(r0, rpt)], out.at[c, pl.ds(r0, rpt)])

    zeros = jnp.zeros((rows_out, HF), F32)
    return pl.kernel(
        body,
        out_type=jax.ShapeDtypeStruct((NC, rows_out, HF), F32),
        mesh=_mesh(),
        scratch_types=(
            [pltpu.VMEM((NCHT // 2, CH), jnp.int32),
             pltpu.VMEM((NCHT // 2, CH), jnp.int32)]
            + [pltpu.VMEM((CH, HF), F32)] * nbuf
            + [pltpu.VMEM_SHARED((rows_out, HF), F32)]
            + [pltpu.SemaphoreType.DMA] * nbuf
        ),
    )(table2, src2.reshape(2 * NNZ_P // CH, CH), dst.reshape(NNZ_P // CH, CH),
      zeros)


def _sc_degrees(dkey):
    rpt = NP // NS

    def body(dkey_r, zeros_r, ones_r, out, dk_b, ones_v, dacc, sem):
        c = lax.axis_index("c")
        s = lax.axis_index("s")
        r0 = s * rpt
        pltpu.sync_copy(zeros_r.at[pl.ds(r0, rpt)], dacc.at[pl.ds(r0, rpt)])
        pltpu.sync_copy(ones_r, ones_v)
        pltpu.sync_copy(
            dkey_r.at[pl.ds(pl.multiple_of((c * NNZ_P + s * EPT) // CH, 8),
                            NCHT)], dk_b)
        plsc.subcore_barrier()

        def chunk(i, carry):
            k0 = i * 4
            for j in range(4):
                pltpu.async_copy(ones_v, dacc.at[dk_b.at[k0 + j]], sem,
                                 add=True)
            for j in range(4):
                pltpu.make_async_copy(ones_v, dacc.at[dk_b.at[k0]],
                                      sem).wait()
            return carry

        lax.fori_loop(0, NCHT // 4, chunk, 0)
        plsc.subcore_barrier()
        pltpu.sync_copy(dacc.at[pl.ds(r0, rpt)], out.at[c, pl.ds(r0, rpt)])

    zeros = jnp.zeros((NP, HF), F32)
    ones = jnp.ones((CH, HF), F32)
    return pl.kernel(
        body,
        out_type=jax.ShapeDtypeStruct((NC, NP, HF), F32),
        mesh=_mesh(),
        scratch_types=[
            pltpu.VMEM((NCHT, CH), jnp.int32),
            pltpu.VMEM((CH, HF), F32),
            pltpu.VMEM_SHARED((NP, HF), F32),
            pltpu.SemaphoreType.DMA,
        ],
    )(dkey.reshape(2 * NNZ_P // CH, CH), zeros, ones)


def kernel(x, hyperedge_index, W_n2e1, W_e2n1, W_n2e2, W_e2n2,
           a1, a2, a_h, num_nodes, num_edges):
    node_idx = hyperedge_index[0]
    edge_idx = hyperedge_index[1]

    pad = jnp.arange(PAD, dtype=jnp.int32)
    node_pad = jnp.concatenate([node_idx, N + pad % (NP - N)])
    edge_pad = jnp.concatenate([edge_idx, E + pad % (EP - E)])
    src_e = jnp.concatenate([2 * node_pad, 2 * node_pad + 1])
    src_n = jnp.concatenate([2 * edge_pad, 2 * edge_pad + 1])

    x_pad = jnp.pad(x, ((0, NP - N), (0, 0)))
    a1r = jnp.reshape(a1, (1, 1)).astype(F32)
    a2r = jnp.reshape(a2, (1, 1)).astype(F32)
    ahr = jnp.reshape(a_h, (1, 1)).astype(F32)

    dego = _sc_degrees(jnp.concatenate([edge_pad, node_pad]))
    deg_e = dego[0, :EP]
    deg_n = dego[1]

    xe1, xn_self1 = _front(x_pad, W_n2e1, W_e2n1, a1r)
    et = _sc_aggregate(xe1.reshape(2 * NP, HF), src_e, edge_pad, EP)
    (xn_top1,) = _edge_stage(et[0], et[1], deg_e, W_e2n1, a1r, want_e=False)
    nn = _sc_aggregate(xn_top1.reshape(2 * EP, HF), src_n, node_pad, NP)

    xe2, xn_self2 = _mid(nn[0], nn[1], xn_self1, deg_n, ahr,
                         W_n2e2, W_e2n2, a2r)
    etb = _sc_aggregate(xe2.reshape(2 * NP, HF), src_e, edge_pad, EP)
    xn_top2, e_out = _edge_stage(etb[0], etb[1], deg_e, W_e2n2, a2r,
                                 want_e=True)
    mm = _sc_aggregate(xn_top2.reshape(2 * EP, HF), src_n, node_pad, NP)
    h2 = _node_stage(mm[0], mm[1], xn_self2, deg_n, ahr)

    return h2[:N], e_out[:E]


# speedup vs baseline: 1.3889x; 1.3889x over previous
import functools

import jax
import jax.numpy as jnp
from jax import lax
from jax.experimental import pallas as pl
from jax.experimental.pallas import tpu as pltpu
from jax.experimental.pallas import tpu_sc as plsc

N = 10000
E = 5000
D = 256
HF = 128
NP = 10240
EP = 5120
NNZ = 160000
NNZ_P = 163840
PAD = NNZ_P - NNZ
CH = 128
NC, NS = 2, 16
EPT = NNZ_P // NS
F32 = jnp.float32


def _mm_body(x_ref, w_ref, o_ref):
    o_ref[...] = jnp.dot(x_ref[...], w_ref[...], preferred_element_type=F32)


def _matmul(x, w, bm=1024):
    m = x.shape[0]
    return pl.pallas_call(
        _mm_body,
        grid=(m // bm,),
        in_specs=[pl.BlockSpec((bm, D), lambda i: (i, 0)),
                  pl.BlockSpec((D, D), lambda i: (0, 0))],
        out_specs=pl.BlockSpec((bm, D), lambda i: (i, 0)),
        out_shape=jax.ShapeDtypeStruct((m, D), F32),
    )(x, w)


def _edge_stage(et0, et1, deg_e, w, a, want_e, bm=1024):
    m = et0.shape[0]

    def body(et0_ref, et1_ref, deg_ref, w_ref, a_ref, *out_refs):
        cnt = deg_ref[:, 0:1]
        s = jnp.where(cnt > 0.0, 1.0 / cnt, 0.0)
        al = a_ref[0, 0]
        e0 = et0_ref[...] * s
        e0 = jnp.where(e0 >= 0.0, e0, al * e0)
        e1 = et1_ref[...] * s
        e1 = jnp.where(e1 >= 0.0, e1, al * e1)
        xn = (jnp.dot(e0, w_ref[:HF, :], preferred_element_type=F32)
              + jnp.dot(e1, w_ref[HF:, :], preferred_element_type=F32))
        out_refs[0][...] = xn
        if want_e:
            out_refs[1][:, :HF] = e0
            out_refs[1][:, HF:] = e1

    n_out = 2 if want_e else 1
    return pl.pallas_call(
        body,
        grid=(m // bm,),
        in_specs=[pl.BlockSpec((bm, HF), lambda i: (i, 0)),
                  pl.BlockSpec((bm, HF), lambda i: (i, 0)),
                  pl.BlockSpec((bm, HF), lambda i: (i, 0)),
                  pl.BlockSpec((D, D), lambda i: (0, 0)),
                  pl.BlockSpec(memory_space=pltpu.SMEM)],
        out_specs=[pl.BlockSpec((bm, D), lambda i: (i, 0))] * n_out,
        out_shape=[jax.ShapeDtypeStruct((m, D), F32)] * n_out,
    )(et0, et1, deg_e, w, a)


def _front(x, w_n2e, w_e2n, a, bm=1024):
    m = x.shape[0]

    def body(x_ref, w1_ref, w2_ref, a_ref, xe_ref, xs_ref):
        al = a_ref[0, 0]
        xe = jnp.dot(x_ref[...], w1_ref[...], preferred_element_type=F32)
        xe_ref[...] = xe
        p = jnp.where(xe >= 0.0, xe, al * xe)
        xs_ref[...] = jnp.dot(p, w2_ref[...], preferred_element_type=F32)

    return pl.pallas_call(
        body,
        grid=(m // bm,),
        in_specs=[pl.BlockSpec((bm, D), lambda i: (i, 0)),
                  pl.BlockSpec((D, D), lambda i: (0, 0)),
                  pl.BlockSpec((D, D), lambda i: (0, 0)),
                  pl.BlockSpec(memory_space=pltpu.SMEM)],
        out_specs=[pl.BlockSpec((bm, D), lambda i: (i, 0))] * 2,
        out_shape=[jax.ShapeDtypeStruct((m, D), F32)] * 2,
    )(x, w_n2e, w_e2n, a)


def _mid(n0, n1, xn_self, deg_n, a_h, w_n2e, w_e2n, a, bm=1024):
    m = n0.shape[0]

    def body(n0_ref, n1_ref, xs_ref, deg_ref, ah_ref, w1_ref, w2_ref, a_ref,
             xe_ref, xs2_ref):
        sc = 1.0 / (deg_ref[:, 0:1] + 1.0)
        ah = ah_ref[0, 0]
        al = a_ref[0, 0]
        t0 = (n0_ref[...] + xs_ref[:, :HF]) * sc
        t0 = jnp.where(t0 >= 0.0, t0, ah * t0)
        t1 = (n1_ref[...] + xs_ref[:, HF:]) * sc
        t1 = jnp.where(t1 >= 0.0, t1, ah * t1)
        xe = (jnp.dot(t0, w1_ref[:HF, :], preferred_element_type=F32)
              + jnp.dot(t1, w1_ref[HF:, :], preferred_element_type=F32))
        xe_ref[...] = xe
        p = jnp.where(xe >= 0.0, xe, al * xe)
        xs2_ref[...] = jnp.dot(p, w2_ref[...], preferred_element_type=F32)

    return pl.pallas_call(
        body,
        grid=(m // bm,),
        in_specs=[pl.BlockSpec((bm, HF), lambda i: (i, 0)),
                  pl.BlockSpec((bm, HF), lambda i: (i, 0)),
                  pl.BlockSpec((bm, D), lambda i: (i, 0)),
                  pl.BlockSpec((bm, HF), lambda i: (i, 0)),
                  pl.BlockSpec(memory_space=pltpu.SMEM),
                  pl.BlockSpec((D, D), lambda i: (0, 0)),
                  pl.BlockSpec((D, D), lambda i: (0, 0)),
                  pl.BlockSpec(memory_space=pltpu.SMEM)],
        out_specs=[pl.BlockSpec((bm, D), lambda i: (i, 0))] * 2,
        out_shape=[jax.ShapeDtypeStruct((m, D), F32)] * 2,
    )(n0, n1, xn_self, deg_n, a_h, w_n2e, w_e2n, a)


def _node_stage(n0, n1, xn_self, deg_n, a_h, bm=1024):
    m = n0.shape[0]

    def body(n0_ref, n1_ref, xs_ref, deg_ref, a_ref, h_ref):
        s = 1.0 / (deg_ref[:, 0:1] + 1.0)
        ah = a_ref[0, 0]
        t0 = (n0_ref[...] + xs_ref[:, :HF]) * s
        t0 = jnp.where(t0 >= 0.0, t0, ah * t0)
        t1 = (n1_ref[...] + xs_ref[:, HF:]) * s
        t1 = jnp.where(t1 >= 0.0, t1, ah * t1)
        h_ref[:, :HF] = t0
        h_ref[:, HF:] = t1

    return pl.pallas_call(
        body,
        grid=(m // bm,),
        in_specs=[pl.BlockSpec((bm, HF), lambda i: (i, 0)),
                  pl.BlockSpec((bm, HF), lambda i: (i, 0)),
                  pl.BlockSpec((bm, D), lambda i: (i, 0)),
                  pl.BlockSpec((bm, HF), lambda i: (i, 0)),
                  pl.BlockSpec(memory_space=pltpu.SMEM)],
        out_specs=pl.BlockSpec((bm, D), lambda i: (i, 0)),
        out_shape=jax.ShapeDtypeStruct((m, D), F32),
    )(n0, n1, xn_self, deg_n, a_h)


@functools.cache
def _mesh():
    return plsc.VectorSubcoreMesh(core_axis_name="c", subcore_axis_name="s",
                                  num_cores=NC, num_subcores=NS)


NCHT = EPT // CH


def _sc_aggregate(table2, src2, dst, rows_out):
    rpt = rows_out // NS
    NH = NCHT // 2

    nbuf = 2

    def body(table2_r, src2_r, dst_r, zeros_r, out, src_b, dst_b, *rest):
        bufs, sems = rest[:nbuf], rest[nbuf + 1:]
        acc = rest[nbuf]
        c = lax.axis_index("c")
        s = lax.axis_index("s")
        r0 = s * rpt
        pltpu.sync_copy(zeros_r.at[pl.ds(r0, rpt)], acc.at[pl.ds(r0, rpt)])
        plsc.subcore_barrier()

        def gather(k, buf, sem):
            pltpu.async_copy(table2_r.at[src_b.at[k]], buf, sem)

        def gwait(buf, sem):
            pltpu.make_async_copy(table2_r.at[src_b.at[0]], buf, sem).wait()

        for h in range(2):
            srow = (c * NNZ_P + s * EPT) // CH + h * NH
            pltpu.sync_copy(
                src2_r.at[pl.ds(pl.multiple_of(srow, 8), NH)], src_b)
            pltpu.sync_copy(
                dst_r.at[pl.ds(pl.multiple_of(s * NCHT + h * NH, 8), NH)],
                dst_b)
            for j in range(nbuf - 1):
                gather(j, bufs[j], sems[j])

            def step(i, carry):
                k0 = i * nbuf
                for j in range(nbuf):
                    gather(k0 + j + nbuf - 1, bufs[(j + nbuf - 1) % nbuf],
                           sems[(j + nbuf - 1) % nbuf])
                    gwait(bufs[j], sems[j])
                    pltpu.sync_copy(bufs[j], acc.at[dst_b.at[k0 + j]],
                                    add=True)
                return carry

            lax.fori_loop(0, NH // nbuf - 1, step, 0)
            k0 = NH - nbuf
            gather(NH - 1, bufs[(nbuf - 1) % nbuf], sems[(nbuf - 1) % nbuf])
            for j in range(nbuf):
                gwait(bufs[j], sems[j])
                pltpu.sync_copy(bufs[j], acc.at[dst_b.at[k0 + j]], add=True)
        plsc.subcore_barrier()
        pltpu.sync_copy(acc.at[pl.ds(r0, rpt)], out.at[c, pl.ds(r0, rpt)])

    zeros = jnp.zeros((rows_out, HF), F32)
    return pl.kernel(
        body,
        out_type=jax.ShapeDtypeStruct((NC, rows_out, HF), F32),
        mesh=_mesh(),
        scratch_types=(
            [pltpu.VMEM((NCHT // 2, CH), jnp.int32),
             pltpu.VMEM((NCHT // 2, CH), jnp.int32)]
            + [pltpu.VMEM((CH, HF), F32)] * nbuf
            + [pltpu.VMEM_SHARED((rows_out, HF), F32)]
            + [pltpu.SemaphoreType.DMA] * nbuf
        ),
    )(table2, src2.reshape(2 * NNZ_P // CH, CH), dst.reshape(NNZ_P // CH, CH),
      zeros)


def _sc_degrees(dkey):
    rpt = NP // NS

    def body(dkey_r, zeros_r, ones_r, out, dk_b, ones_v, dacc, sem):
        c = lax.axis_index("c")
        s = lax.axis_index("s")
        r0 = s * rpt
        pltpu.sync_copy(zeros_r.at[pl.ds(r0, rpt)], dacc.at[pl.ds(r0, rpt)])
        pltpu.sync_copy(ones_r, ones_v)
        pltpu.sync_copy(
            dkey_r.at[pl.ds(pl.multiple_of((c * NNZ_P + s * EPT) // CH, 8),
                            NCHT)], dk_b)
        plsc.subcore_barrier()

        def chunk(i, carry):
            k0 = i * 4
            for j in range(4):
                pltpu.async_copy(ones_v, dacc.at[dk_b.at[k0 + j]], sem,
                                 add=True)
            for j in range(4):
                pltpu.make_async_copy(ones_v, dacc.at[dk_b.at[k0]],
                                      sem).wait()
            return carry

        lax.fori_loop(0, NCHT // 4, chunk, 0)
        plsc.subcore_barrier()
        pltpu.sync_copy(dacc.at[pl.ds(r0, rpt)], out.at[c, pl.ds(r0, rpt)])

    zeros = jnp.zeros((NP, HF), F32)
    ones = jnp.ones((CH, HF), F32)
    return pl.kernel(
        body,
        out_type=jax.ShapeDtypeStruct((NC, NP, HF), F32),
        mesh=_mesh(),
        scratch_types=[
            pltpu.VMEM((NCHT, CH), jnp.int32),
            pltpu.VMEM((CH, HF), F32),
            pltpu.VMEM_SHARED((NP, HF), F32),
            pltpu.SemaphoreType.DMA,
        ],
    )(dkey.reshape(2 * NNZ_P // CH, CH), zeros, ones)


def kernel(x, hyperedge_index, W_n2e1, W_e2n1, W_n2e2, W_e2n2,
           a1, a2, a_h, num_nodes, num_edges):
    node_idx = hyperedge_index[0]
    edge_idx = hyperedge_index[1]

    pad = jnp.arange(PAD, dtype=jnp.int32)
    node_pad = jnp.concatenate([node_idx, N + pad % (NP - N)])
    edge_pad = jnp.concatenate([edge_idx, E + pad % (EP - E)])
    src_e = jnp.concatenate([2 * node_pad, 2 * node_pad + 1])
    src_n = jnp.concatenate([2 * edge_pad, 2 * edge_pad + 1])

    x_pad = jnp.pad(x, ((0, NP - N), (0, 0)))
    a1r = jnp.reshape(a1, (1, 1)).astype(F32)
    a2r = jnp.reshape(a2, (1, 1)).astype(F32)
    ahr = jnp.reshape(a_h, (1, 1)).astype(F32)

    dego = _sc_degrees(jnp.concatenate([edge_pad, node_pad]))
    deg_e = dego[0, :EP]
    deg_n = dego[1]

    xe1, xn_self1 = _front(x_pad, W_n2e1, W_e2n1, a1r)
    et = _sc_aggregate(xe1.reshape(2 * NP, HF), src_e, edge_pad, EP)
    (xn_top1,) = _edge_stage(et[0], et[1], deg_e, W_e2n1, a1r, want_e=False)
    nn = _sc_aggregate(xn_top1.reshape(2 * EP, HF), src_n, node_pad, NP)

    xe2, xn_self2 = _mid(nn[0], nn[1], xn_self1, deg_n, ahr,
                         W_n2e2, W_e2n2, a2r)
    etb = _sc_aggregate(xe2.reshape(2 * NP, HF), src_e, edge_pad, EP)
    xn_top2, e_out = _edge_stage(etb[0], etb[1], deg_e, W_e2n2, a2r,
                                 want_e=True)
    mm = _sc_aggregate(xn_top2.reshape(2 * EP, HF), src_n, node_pad, NP)
    h2 = _node_stage(mm[0], mm[1], xn_self2, deg_n, ahr)

    return h2[:N], e_out[:E]


# speedup vs baseline: 1.4324x; 1.0313x over previous
import functools

import jax
import jax.numpy as jnp
from jax import lax
from jax.experimental import pallas as pl
from jax.experimental.pallas import tpu as pltpu
from jax.experimental.pallas import tpu_sc as plsc

N = 10000
E = 5000
D = 256
HF = 128
NP = 10240
EP = 5120
NNZ = 160000
NNZ_P = 163840
PAD = NNZ_P - NNZ
CH = 128
NC, NS = 2, 16
EPT = NNZ_P // NS
F32 = jnp.float32


def _mm_body(x_ref, w_ref, o_ref):
    o_ref[...] = jnp.dot(x_ref[...], w_ref[...], preferred_element_type=F32)


def _matmul(x, w, bm=1024):
    m = x.shape[0]
    return pl.pallas_call(
        _mm_body,
        grid=(m // bm,),
        in_specs=[pl.BlockSpec((bm, D), lambda i: (i, 0)),
                  pl.BlockSpec((D, D), lambda i: (0, 0))],
        out_specs=pl.BlockSpec((bm, D), lambda i: (i, 0)),
        out_shape=jax.ShapeDtypeStruct((m, D), F32),
    )(x, w)


def _edge_stage(et0, et1, deg_e, w, a, want_e, bm=1024):
    m = et0.shape[0]

    def body(et0_ref, et1_ref, deg_ref, w_ref, a_ref, *out_refs):
        cnt = deg_ref[:, 0:1]
        s = jnp.where(cnt > 0.0, 1.0 / cnt, 0.0)
        al = a_ref[0, 0]
        e0 = et0_ref[...] * s
        e0 = jnp.where(e0 >= 0.0, e0, al * e0)
        e1 = et1_ref[...] * s
        e1 = jnp.where(e1 >= 0.0, e1, al * e1)
        xn = (jnp.dot(e0, w_ref[:HF, :], preferred_element_type=F32)
              + jnp.dot(e1, w_ref[HF:, :], preferred_element_type=F32))
        out_refs[0][...] = xn
        if want_e:
            out_refs[1][:, :HF] = e0
            out_refs[1][:, HF:] = e1

    n_out = 2 if want_e else 1
    return pl.pallas_call(
        body,
        grid=(m // bm,),
        in_specs=[pl.BlockSpec((bm, HF), lambda i: (i, 0)),
                  pl.BlockSpec((bm, HF), lambda i: (i, 0)),
                  pl.BlockSpec((bm, HF), lambda i: (i, 0)),
                  pl.BlockSpec((D, D), lambda i: (0, 0)),
                  pl.BlockSpec(memory_space=pltpu.SMEM)],
        out_specs=[pl.BlockSpec((bm, D), lambda i: (i, 0))] * n_out,
        out_shape=[jax.ShapeDtypeStruct((m, D), F32)] * n_out,
    )(et0, et1, deg_e, w, a)


def _front(x, w_n2e, w_e2n, a, bm=1024):
    m = x.shape[0]

    def body(x_ref, w1_ref, w2_ref, a_ref, xe_ref, xs_ref):
        al = a_ref[0, 0]
        xe = jnp.dot(x_ref[...], w1_ref[...], preferred_element_type=F32)
        xe_ref[...] = xe
        p = jnp.where(xe >= 0.0, xe, al * xe)
        xs_ref[...] = jnp.dot(p, w2_ref[...], preferred_element_type=F32)

    return pl.pallas_call(
        body,
        grid=(m // bm,),
        in_specs=[pl.BlockSpec((bm, D), lambda i: (i, 0)),
                  pl.BlockSpec((D, D), lambda i: (0, 0)),
                  pl.BlockSpec((D, D), lambda i: (0, 0)),
                  pl.BlockSpec(memory_space=pltpu.SMEM)],
        out_specs=[pl.BlockSpec((bm, D), lambda i: (i, 0))] * 2,
        out_shape=[jax.ShapeDtypeStruct((m, D), F32)] * 2,
    )(x, w_n2e, w_e2n, a)


def _mid(n0, n1, xn_self, deg_n, a_h, w_n2e, w_e2n, a, bm=1024):
    m = n0.shape[0]

    def body(n0_ref, n1_ref, xs_ref, deg_ref, ah_ref, w1_ref, w2_ref, a_ref,
             xe_ref, xs2_ref):
        sc = 1.0 / (deg_ref[:, 0:1] + 1.0)
        ah = ah_ref[0, 0]
        al = a_ref[0, 0]
        t0 = (n0_ref[...] + xs_ref[:, :HF]) * sc
        t0 = jnp.where(t0 >= 0.0, t0, ah * t0)
        t1 = (n1_ref[...] + xs_ref[:, HF:]) * sc
        t1 = jnp.where(t1 >= 0.0, t1, ah * t1)
        xe = (jnp.dot(t0, w1_ref[:HF, :], preferred_element_type=F32)
              + jnp.dot(t1, w1_ref[HF:, :], preferred_element_type=F32))
        xe_ref[...] = xe
        p = jnp.where(xe >= 0.0, xe, al * xe)
        xs2_ref[...] = jnp.dot(p, w2_ref[...], preferred_element_type=F32)

    return pl.pallas_call(
        body,
        grid=(m // bm,),
        in_specs=[pl.BlockSpec((bm, HF), lambda i: (i, 0)),
                  pl.BlockSpec((bm, HF), lambda i: (i, 0)),
                  pl.BlockSpec((bm, D), lambda i: (i, 0)),
                  pl.BlockSpec((bm, HF), lambda i: (i, 0)),
                  pl.BlockSpec(memory_space=pltpu.SMEM),
                  pl.BlockSpec((D, D), lambda i: (0, 0)),
                  pl.BlockSpec((D, D), lambda i: (0, 0)),
                  pl.BlockSpec(memory_space=pltpu.SMEM)],
        out_specs=[pl.BlockSpec((bm, D), lambda i: (i, 0))] * 2,
        out_shape=[jax.ShapeDtypeStruct((m, D), F32)] * 2,
    )(n0, n1, xn_self, deg_n, a_h, w_n2e, w_e2n, a)


def _node_stage(n0, n1, xn_self, deg_n, a_h, bm=1024):
    m = n0.shape[0]

    def body(n0_ref, n1_ref, xs_ref, deg_ref, a_ref, h_ref):
        s = 1.0 / (deg_ref[:, 0:1] + 1.0)
        ah = a_ref[0, 0]
        t0 = (n0_ref[...] + xs_ref[:, :HF]) * s
        t0 = jnp.where(t0 >= 0.0, t0, ah * t0)
        t1 = (n1_ref[...] + xs_ref[:, HF:]) * s
        t1 = jnp.where(t1 >= 0.0, t1, ah * t1)
        h_ref[:, :HF] = t0
        h_ref[:, HF:] = t1

    return pl.pallas_call(
        body,
        grid=(m // bm,),
        in_specs=[pl.BlockSpec((bm, HF), lambda i: (i, 0)),
                  pl.BlockSpec((bm, HF), lambda i: (i, 0)),
                  pl.BlockSpec((bm, D), lambda i: (i, 0)),
                  pl.BlockSpec((bm, HF), lambda i: (i, 0)),
                  pl.BlockSpec(memory_space=pltpu.SMEM)],
        out_specs=pl.BlockSpec((bm, D), lambda i: (i, 0)),
        out_shape=jax.ShapeDtypeStruct((m, D), F32),
    )(n0, n1, xn_self, deg_n, a_h)


@functools.cache
def _mesh():
    return plsc.VectorSubcoreMesh(core_axis_name="c", subcore_axis_name="s",
                                  num_cores=NC, num_subcores=NS)


NCHT = EPT // CH


def _sc_aggregate(table2, src2, dst, rows_out):
    rpt = rows_out // NS
    NH = NCHT // 2

    nbuf = 4 if rows_out <= EP else 2

    def body(table2_r, src2_r, dst_r, zeros_r, out, src_b, dst_b, *rest):
        bufs, sems = rest[:nbuf], rest[nbuf + 1:]
        acc = rest[nbuf]
        c = lax.axis_index("c")
        s = lax.axis_index("s")
        r0 = s * rpt
        pltpu.sync_copy(zeros_r.at[pl.ds(r0, rpt)], acc.at[pl.ds(r0, rpt)])
        plsc.subcore_barrier()

        def gather(k, buf, sem):
            pltpu.async_copy(table2_r.at[src_b.at[k]], buf, sem)

        def gwait(buf, sem):
            pltpu.make_async_copy(table2_r.at[src_b.at[0]], buf, sem).wait()

        for h in range(2):
            srow = (c * NNZ_P + s * EPT) // CH + h * NH
            pltpu.sync_copy(
                src2_r.at[pl.ds(pl.multiple_of(srow, 8), NH)], src_b)
            pltpu.sync_copy(
                dst_r.at[pl.ds(pl.multiple_of(s * NCHT + h * NH, 8), NH)],
                dst_b)
            for j in range(nbuf - 1):
                gather(j, bufs[j], sems[j])

            def step(i, carry):
                k0 = i * nbuf
                for j in range(nbuf):
                    gather(k0 + j + nbuf - 1, bufs[(j + nbuf - 1) % nbuf],
                           sems[(j + nbuf - 1) % nbuf])
                    gwait(bufs[j], sems[j])
                    pltpu.sync_copy(bufs[j], acc.at[dst_b.at[k0 + j]],
                                    add=True)
                return carry

            lax.fori_loop(0, NH // nbuf - 1, step, 0)
            k0 = NH - nbuf
            gather(NH - 1, bufs[(nbuf - 1) % nbuf], sems[(nbuf - 1) % nbuf])
            for j in range(nbuf):
                gwait(bufs[j], sems[j])
                pltpu.sync_copy(bufs[j], acc.at[dst_b.at[k0 + j]], add=True)
        plsc.subcore_barrier()
        pltpu.sync_copy(acc.at[pl.ds(r0, rpt)], out.at[c, pl.ds(r0, rpt)])

    zeros = jnp.zeros((rows_out, HF), F32)
    return pl.kernel(
        body,
        out_type=jax.ShapeDtypeStruct((NC, rows_out, HF), F32),
        mesh=_mesh(),
        scratch_types=(
            [pltpu.VMEM((NCHT // 2, CH), jnp.int32),
             pltpu.VMEM((NCHT // 2, CH), jnp.int32)]
            + [pltpu.VMEM((CH, HF), F32)] * nbuf
            + [pltpu.VMEM_SHARED((rows_out, HF), F32)]
            + [pltpu.SemaphoreType.DMA] * nbuf
        ),
    )(table2, src2.reshape(2 * NNZ_P // CH, CH), dst.reshape(NNZ_P // CH, CH),
      zeros)


def _sc_degrees(dkey):
    rpt = NP // NS

    def body(dkey_r, zeros_r, ones_r, out, dk_b, ones_v, dacc, sem):
        c = lax.axis_index("c")
        s = lax.axis_index("s")
        r0 = s * rpt
        pltpu.sync_copy(zeros_r.at[pl.ds(r0, rpt)], dacc.at[pl.ds(r0, rpt)])
        pltpu.sync_copy(ones_r, ones_v)
        pltpu.sync_copy(
            dkey_r.at[pl.ds(pl.multiple_of((c * NNZ_P + s * EPT) // CH, 8),
                            NCHT)], dk_b)
        plsc.subcore_barrier()

        def chunk(i, carry):
            k0 = i * 4
            for j in range(4):
                pltpu.async_copy(ones_v, dacc.at[dk_b.at[k0 + j]], sem,
                                 add=True)
            for j in range(4):
                pltpu.make_async_copy(ones_v, dacc.at[dk_b.at[k0]],
                                      sem).wait()
            return carry

        lax.fori_loop(0, NCHT // 4, chunk, 0)
        plsc.subcore_barrier()
        pltpu.sync_copy(dacc.at[pl.ds(r0, rpt)], out.at[c, pl.ds(r0, rpt)])

    zeros = jnp.zeros((NP, HF), F32)
    ones = jnp.ones((CH, HF), F32)
    return pl.kernel(
        body,
        out_type=jax.ShapeDtypeStruct((NC, NP, HF), F32),
        mesh=_mesh(),
        scratch_types=[
            pltpu.VMEM((NCHT, CH), jnp.int32),
            pltpu.VMEM((CH, HF), F32),
            pltpu.VMEM_SHARED((NP, HF), F32),
            pltpu.SemaphoreType.DMA,
        ],
    )(dkey.reshape(2 * NNZ_P // CH, CH), zeros, ones)


def kernel(x, hyperedge_index, W_n2e1, W_e2n1, W_n2e2, W_e2n2,
           a1, a2, a_h, num_nodes, num_edges):
    node_idx = hyperedge_index[0]
    edge_idx = hyperedge_index[1]

    pad = jnp.arange(PAD, dtype=jnp.int32)
    node_pad = jnp.concatenate([node_idx, N + pad % (NP - N)])
    edge_pad = jnp.concatenate([edge_idx, E + pad % (EP - E)])
    src_e = jnp.concatenate([2 * node_pad, 2 * node_pad + 1])
    src_n = jnp.concatenate([2 * edge_pad, 2 * edge_pad + 1])

    x_pad = jnp.pad(x, ((0, NP - N), (0, 0)))
    a1r = jnp.reshape(a1, (1, 1)).astype(F32)
    a2r = jnp.reshape(a2, (1, 1)).astype(F32)
    ahr = jnp.reshape(a_h, (1, 1)).astype(F32)

    dego = _sc_degrees(jnp.concatenate([edge_pad, node_pad]))
    deg_e = dego[0, :EP]
    deg_n = dego[1]

    xe1, xn_self1 = _front(x_pad, W_n2e1, W_e2n1, a1r)
    et = _sc_aggregate(xe1.reshape(2 * NP, HF), src_e, edge_pad, EP)
    (xn_top1,) = _edge_stage(et[0], et[1], deg_e, W_e2n1, a1r, want_e=False)
    nn = _sc_aggregate(xn_top1.reshape(2 * EP, HF), src_n, node_pad, NP)

    xe2, xn_self2 = _mid(nn[0], nn[1], xn_self1, deg_n, ahr,
                         W_n2e2, W_e2n2, a2r)
    etb = _sc_aggregate(xe2.reshape(2 * NP, HF), src_e, edge_pad, EP)
    xn_top2, e_out = _edge_stage(etb[0], etb[1], deg_e, W_e2n2, a2r,
                                 want_e=True)
    mm = _sc_aggregate(xn_top2.reshape(2 * EP, HF), src_n, node_pad, NP)
    h2 = _node_stage(mm[0], mm[1], xn_self2, deg_n, ahr)

    return h2[:N], e_out[:E]
